# baseline (speedup 1.0000x reference)
import jax
import jax.numpy as jnp
from jax import lax
from jax.experimental import pallas as pl
from jax.experimental.pallas import tpu as pltpu
from jax.experimental.pallas import tpu_sc as plsc

DIM_ = 1024
NSTEPS_ = 3072
ROW_BLOCK = 512
NCORES = 2
NSUB = 16
COLS_PER = DIM_ // NSUB
STEPS_PER = NSTEPS_ // NCORES


def _cs_kernel(a_ref, o_ref):
    a = a_ref[...]
    o_ref[0, :, :] = jnp.cos(a)
    o_ref[1, :, :] = jnp.sin(a)


def _sc_build_body(pi_hbm, pj_hbm, cos_hbm, sin_hbm, out_hbm,
                   pi_v, pj_v, cos_v, sin_v, m_local):
    cid = lax.axis_index("c")
    sid = lax.axis_index("s")
    base = cid * STEPS_PER
    col0 = sid * COLS_PER

    pltpu.sync_copy(pi_hbm.at[pl.ds(base, STEPS_PER)], pi_v)
    pltpu.sync_copy(pj_hbm.at[pl.ds(base, STEPS_PER)], pj_v)
    pltpu.sync_copy(cos_hbm, cos_v)
    pltpu.sync_copy(sin_hbm, sin_v)

    zeros = jnp.zeros((16,), jnp.float32)

    def zero_row(r, _):
        for k in range(COLS_PER // 16):
            m_local[r, pl.ds(16 * k, 16)] = zeros
        return 0

    lax.fori_loop(0, DIM_, zero_row, 0)
    lanes = lax.iota(jnp.int32, 16)
    for q in range(COLS_PER):
        onehot = jnp.where(lanes == (q % 16), 1.0, 0.0).astype(jnp.float32)
        m_local[col0 + q, pl.ds(16 * (q // 16), 16)] = onehot

    def chunk_body(tc, _):
        t0 = tc * 16
        tm0 = lax.rem(base + t0, DIM_)
        pi_c = pi_v[pl.ds(t0, 16)]
        pj_c = pj_v[pl.ds(t0, 16)]
        cos_c = cos_v[pl.ds(tm0, 16)]
        sin_c = sin_v[pl.ds(tm0, 16)]
        for u in range(16):
            i = pi_c[u]
            j = pj_c[u]
            c = cos_c[u]
            s = sin_c[u]
            for k in range(COLS_PER // 16):
                sl = pl.ds(16 * k, 16)
                mi = m_local[i, sl]
                mj = m_local[j, sl]
                m_local[i, sl] = c * mi - s * mj
                m_local[j, sl] = s * mi + c * mj
        return 0

    lax.fori_loop(0, STEPS_PER // 16, chunk_body, 0)

    pltpu.sync_copy(m_local, out_hbm.at[cid, :, pl.ds(col0, COLS_PER)])


def _combine_kernel(a_ref, b_ref, o_ref):
    o_ref[...] = jnp.dot(
        b_ref[...], a_ref[...], preferred_element_type=jnp.float32
    )


def _matmul_kernel(x_ref, m_ref, o_ref):
    o_ref[...] = jax.lax.dot_general(
        x_ref[...], m_ref[...],
        dimension_numbers=(((1,), (1,)), ((), ())),
        preferred_element_type=jnp.float32,
    )


def kernel(x, angles, pairs_i, pairs_j):
    dim = angles.shape[0]
    x2d = x.reshape(-1, dim).astype(jnp.float32)
    n_rows = x2d.shape[0]

    cs = pl.pallas_call(
        _cs_kernel,
        out_shape=jax.ShapeDtypeStruct((2, 8, 128), jnp.float32),
    )(angles.reshape(8, 128).astype(jnp.float32))
    cs = cs.reshape(2, dim)

    mesh = plsc.VectorSubcoreMesh(
        core_axis_name="c", subcore_axis_name="s",
        num_cores=NCORES, num_subcores=NSUB,
    )
    sc_build = pl.kernel(
        _sc_build_body,
        out_type=jax.ShapeDtypeStruct((NCORES, dim, dim), jnp.float32),
        mesh=mesh,
        scratch_types=[
            pltpu.VMEM((STEPS_PER,), jnp.int32),
            pltpu.VMEM((STEPS_PER,), jnp.int32),
            pltpu.VMEM((dim,), jnp.float32),
            pltpu.VMEM((dim,), jnp.float32),
            pltpu.VMEM((dim, COLS_PER), jnp.float32),
        ],
        compiler_params=pltpu.CompilerParams(use_tc_tiling_on_sc=False),
    )
    halves = sc_build(pairs_i, pairs_j, cs[0], cs[1])

    c = pl.pallas_call(
        _combine_kernel,
        out_shape=jax.ShapeDtypeStruct((dim, dim), jnp.float32),
    )(halves[0], halves[1])

    grid = (n_rows // ROW_BLOCK,)
    y2d = pl.pallas_call(
        _matmul_kernel,
        out_shape=jax.ShapeDtypeStruct((n_rows, dim), jnp.float32),
        grid=grid,
        in_specs=[
            pl.BlockSpec((ROW_BLOCK, dim), lambda r: (r, 0)),
            pl.BlockSpec((dim, dim), lambda r: (0, 0)),
        ],
        out_specs=pl.BlockSpec((ROW_BLOCK, dim), lambda r: (r, 0)),
    )(x2d, c)

    return y2d.reshape(x.shape).astype(x.dtype)
